# baseline (speedup 1.0000x reference)
import functools

import jax
import jax.numpy as jnp
import numpy as np
from jax import lax
from jax.experimental import pallas as pl
from jax.experimental.pallas import tpu as pltpu
from jax.experimental.pallas import tpu_sc as plsc

_N_NODES = 10000
_N_EDGES = 320000
_D_FEAT = 128
_SCALAR_DIM = 64
_R_PREV = 16
_OUT_DIM = _R_PREV + 2 * _SCALAR_DIM

_CHUNK = 128
_KSUB = 2
_SUPER = _KSUB * _CHUNK
_NC = 2
_NS = 16
_NW = _NC * _NS

_NB = 10
_EB = _N_EDGES // _NB
_SUPERS_B = _EB // _SUPER
_ITERS_B = (_SUPERS_B + _NW - 1) // _NW

_INV_SQRT_FAN_IN = np.float32(1.0 / np.sqrt(np.float32(_D_FEAT)))


def _matmul_body(x_ref, w_ref, o_ref):
    o_ref[...] = jax.lax.dot_general(
        x_ref[...], w_ref[...],
        dimension_numbers=(((1,), (0,)), ((), ())),
        preferred_element_type=jnp.float32,
    ) * _INV_SQRT_FAN_IN


_node_linear = pl.pallas_call(
    _matmul_body,
    out_shape=jax.ShapeDtypeStruct((_N_NODES, _SCALAR_DIM), jnp.float32),
)


def _gather_body(base, feat, esrc, edst, gb, isrc_v, idst_v, rsrc_v, rdst_v,
                 sem_idx, sem_g, sem_wr):
    wid = lax.axis_index("s") * _NC + lax.axis_index("c")

    def body(s, carry):
        sid = s * _NW + wid

        @pl.when(sid < _SUPERS_B)
        def _():
            r0 = sid * _SUPER
            e0 = base + r0
            cin = []
            for j in range(_KSUB):
                cin.append(pltpu.make_async_copy(
                    esrc.at[pl.ds(e0 + j * _CHUNK, _CHUNK)], isrc_v.at[j], sem_idx))
                cin.append(pltpu.make_async_copy(
                    edst.at[pl.ds(e0 + j * _CHUNK, _CHUNK)], idst_v.at[j], sem_idx))
            for c in cin:
                c.start()
            for c in cin:
                c.wait()
            cg = []
            for j in range(_KSUB):
                rows = pl.ds(j * _CHUNK, _CHUNK)
                cg.append(pltpu.make_async_copy(
                    feat.at[isrc_v.at[j]], rsrc_v.at[rows, :], sem_g))
                cg.append(pltpu.make_async_copy(
                    feat.at[idst_v.at[j]], rdst_v.at[rows, :], sem_g))
            for c in cg:
                c.start()
            for c in cg:
                c.wait()
            cw = [
                pltpu.make_async_copy(
                    rsrc_v, gb.at[pl.ds(r0, _SUPER), pl.ds(0, _SCALAR_DIM)], sem_wr),
                pltpu.make_async_copy(
                    rdst_v, gb.at[pl.ds(r0, _SUPER), pl.ds(_SCALAR_DIM, _SCALAR_DIM)], sem_wr),
            ]
            for c in cw:
                c.start()
            for c in cw:
                c.wait()

        return carry

    lax.fori_loop(0, _ITERS_B, body, 0)


def _make_gather(b):
    return functools.partial(
        pl.kernel,
        out_type=jax.ShapeDtypeStruct((_EB, 2 * _SCALAR_DIM), jnp.float32),
        mesh=plsc.VectorSubcoreMesh(
            core_axis_name="c", subcore_axis_name="s", num_cores=_NC, num_subcores=_NS
        ),
        scratch_types=[
            pltpu.VMEM((_KSUB, _CHUNK), jnp.int32),
            pltpu.VMEM((_KSUB, _CHUNK), jnp.int32),
            pltpu.VMEM((_SUPER, _SCALAR_DIM), jnp.float32),
            pltpu.VMEM((_SUPER, _SCALAR_DIM), jnp.float32),
            pltpu.SemaphoreType.DMA,
            pltpu.SemaphoreType.DMA,
            pltpu.SemaphoreType.DMA,
        ],
        compiler_params=pltpu.CompilerParams(use_tc_tiling_on_sc=False),
    )(functools.partial(_gather_body, b * _EB))


_gathers = [_make_gather(b) for b in range(_NB)]


_CB = 3200
_GSTEPS = _EB // _CB


def _concat_body(p_ref, b_ref, o_ref):
    o_ref[0:_R_PREV, :] = p_ref[...]
    o_ref[_R_PREV:_OUT_DIM, :] = b_ref[...].T


def _concat_body_alias(a_ref, p_ref, b_ref, o_ref):
    del a_ref
    o_ref[0:_R_PREV, :] = p_ref[...]
    o_ref[_R_PREV:_OUT_DIM, :] = b_ref[...].T


def _make_concat(b):
    base = b * _GSTEPS
    if b == 0:
        return pl.pallas_call(
            _concat_body,
            grid=(_GSTEPS,),
            in_specs=[
                pl.BlockSpec((_R_PREV, _CB), lambda i: (0, base + i)),
                pl.BlockSpec((_CB, 2 * _SCALAR_DIM), lambda i: (i, 0)),
            ],
            out_specs=pl.BlockSpec((_OUT_DIM, _CB), lambda i: (0, base + i)),
            out_shape=jax.ShapeDtypeStruct((_OUT_DIM, _N_EDGES), jnp.float32),
        )
    return pl.pallas_call(
        _concat_body_alias,
        grid=(_GSTEPS,),
        in_specs=[
            pl.BlockSpec(memory_space=pl.ANY),
            pl.BlockSpec((_R_PREV, _CB), lambda i: (0, base + i)),
            pl.BlockSpec((_CB, 2 * _SCALAR_DIM), lambda i: (i, 0)),
        ],
        out_specs=pl.BlockSpec((_OUT_DIM, _CB), lambda i: (0, base + i)),
        out_shape=jax.ShapeDtypeStruct((_OUT_DIM, _N_EDGES), jnp.float32),
        input_output_aliases={0: 0},
    )


_concats = [_make_concat(b) for b in range(_NB)]


@jax.jit
def kernel(x, radial_weights_prev, edge_index, W):
    feat = _node_linear(x, W)
    edge_src = edge_index[1]
    edge_dst = edge_index[0]
    prev_t = radial_weights_prev.T
    gbs = [_gathers[b](feat, edge_src, edge_dst) for b in range(_NB)]
    out_t = _concats[0](prev_t, gbs[0])
    for b in range(1, _NB):
        out_t = _concats[b](out_t, prev_t, gbs[b])
    return out_t.T


# speedup vs baseline: 1.0659x; 1.0659x over previous
import functools

import jax
import jax.numpy as jnp
import numpy as np
from jax import lax
from jax.experimental import pallas as pl
from jax.experimental.pallas import tpu as pltpu
from jax.experimental.pallas import tpu_sc as plsc

_N_NODES = 10000
_N_EDGES = 320000
_D_FEAT = 128
_SCALAR_DIM = 64
_R_PREV = 16
_OUT_DIM = _R_PREV + 2 * _SCALAR_DIM

_CHUNK = 128
_KSUB = 4
_SUPER = _KSUB * _CHUNK
_NC = 2
_NS = 16
_NW = _NC * _NS

_UNIT = 12800
_BLOCK_UNITS = [2, 6, 6, 6, 4, 1]
_BLOCK_EDGES = [u * _UNIT for u in _BLOCK_UNITS]
_BLOCK_BASE = [sum(_BLOCK_EDGES[:b]) for b in range(len(_BLOCK_EDGES))]
_NB = len(_BLOCK_EDGES)

_INV_SQRT_FAN_IN = np.float32(1.0 / np.sqrt(np.float32(_D_FEAT)))


def _matmul_body(x_ref, w_ref, o_ref):
    o_ref[...] = jax.lax.dot_general(
        x_ref[...], w_ref[...],
        dimension_numbers=(((1,), (0,)), ((), ())),
        preferred_element_type=jnp.float32,
    ) * _INV_SQRT_FAN_IN


_node_linear = pl.pallas_call(
    _matmul_body,
    out_shape=jax.ShapeDtypeStruct((_N_NODES, _SCALAR_DIM), jnp.float32),
)


def _gather_body(base, n_supers, feat, esrc, edst, gb, isrc_v, idst_v, rsrc_v,
                 rdst_v, sem_idx, sem_g, sem_wr):
    wid = lax.axis_index("s") * _NC + lax.axis_index("c")

    def body(s, carry):
        sid = s * _NW + wid

        @pl.when(sid < n_supers)
        def _():
            r0 = sid * _SUPER
            e0 = base + r0
            cin = []
            for j in range(_KSUB):
                cin.append(pltpu.make_async_copy(
                    esrc.at[pl.ds(e0 + j * _CHUNK, _CHUNK)], isrc_v.at[j], sem_idx))
                cin.append(pltpu.make_async_copy(
                    edst.at[pl.ds(e0 + j * _CHUNK, _CHUNK)], idst_v.at[j], sem_idx))
            for c in cin:
                c.start()
            for c in cin:
                c.wait()
            cg = []
            for j in range(_KSUB):
                rows = pl.ds(j * _CHUNK, _CHUNK)
                cg.append(pltpu.make_async_copy(
                    feat.at[isrc_v.at[j]], rsrc_v.at[rows, :], sem_g))
                cg.append(pltpu.make_async_copy(
                    feat.at[idst_v.at[j]], rdst_v.at[rows, :], sem_g))
            for c in cg:
                c.start()
            for c in cg:
                c.wait()
            cw = [
                pltpu.make_async_copy(
                    rsrc_v, gb.at[pl.ds(r0, _SUPER), pl.ds(0, _SCALAR_DIM)], sem_wr),
                pltpu.make_async_copy(
                    rdst_v, gb.at[pl.ds(r0, _SUPER), pl.ds(_SCALAR_DIM, _SCALAR_DIM)], sem_wr),
            ]
            for c in cw:
                c.start()
            for c in cw:
                c.wait()

        return carry

    iters = (n_supers + _NW - 1) // _NW
    lax.fori_loop(0, iters, body, 0)


def _make_gather(b):
    n_supers = _BLOCK_EDGES[b] // _SUPER
    return functools.partial(
        pl.kernel,
        out_type=jax.ShapeDtypeStruct((_BLOCK_EDGES[b], 2 * _SCALAR_DIM), jnp.float32),
        mesh=plsc.VectorSubcoreMesh(
            core_axis_name="c", subcore_axis_name="s", num_cores=_NC, num_subcores=_NS
        ),
        scratch_types=[
            pltpu.VMEM((_KSUB, _CHUNK), jnp.int32),
            pltpu.VMEM((_KSUB, _CHUNK), jnp.int32),
            pltpu.VMEM((_SUPER, _SCALAR_DIM), jnp.float32),
            pltpu.VMEM((_SUPER, _SCALAR_DIM), jnp.float32),
            pltpu.SemaphoreType.DMA,
            pltpu.SemaphoreType.DMA,
            pltpu.SemaphoreType.DMA,
        ],
        compiler_params=pltpu.CompilerParams(use_tc_tiling_on_sc=False),
    )(functools.partial(_gather_body, _BLOCK_BASE[b], n_supers))


_gathers = [_make_gather(b) for b in range(_NB)]


_CB = 3200


def _concat_body(p_ref, b_ref, o_ref):
    o_ref[0:_R_PREV, :] = p_ref[...]
    o_ref[_R_PREV:_OUT_DIM, :] = b_ref[...].T


def _concat_body_alias(a_ref, p_ref, b_ref, o_ref):
    del a_ref
    o_ref[0:_R_PREV, :] = p_ref[...]
    o_ref[_R_PREV:_OUT_DIM, :] = b_ref[...].T


def _make_concat(b):
    base = _BLOCK_BASE[b] // _CB
    gsteps = _BLOCK_EDGES[b] // _CB
    if b == 0:
        return pl.pallas_call(
            _concat_body,
            grid=(gsteps,),
            in_specs=[
                pl.BlockSpec((_R_PREV, _CB), lambda i: (0, base + i)),
                pl.BlockSpec((_CB, 2 * _SCALAR_DIM), lambda i: (i, 0)),
            ],
            out_specs=pl.BlockSpec((_OUT_DIM, _CB), lambda i: (0, base + i)),
            out_shape=jax.ShapeDtypeStruct((_OUT_DIM, _N_EDGES), jnp.float32),
        )
    return pl.pallas_call(
        _concat_body_alias,
        grid=(gsteps,),
        in_specs=[
            pl.BlockSpec(memory_space=pl.ANY),
            pl.BlockSpec((_R_PREV, _CB), lambda i: (0, base + i)),
            pl.BlockSpec((_CB, 2 * _SCALAR_DIM), lambda i: (i, 0)),
        ],
        out_specs=pl.BlockSpec((_OUT_DIM, _CB), lambda i: (0, base + i)),
        out_shape=jax.ShapeDtypeStruct((_OUT_DIM, _N_EDGES), jnp.float32),
        input_output_aliases={0: 0},
    )


_concats = [_make_concat(b) for b in range(_NB)]


@jax.jit
def kernel(x, radial_weights_prev, edge_index, W):
    feat = _node_linear(x, W)
    edge_src = edge_index[1]
    edge_dst = edge_index[0]
    prev_t = radial_weights_prev.T
    gbs = [_gathers[b](feat, edge_src, edge_dst) for b in range(_NB)]
    out_t = _concats[0](prev_t, gbs[0])
    for b in range(1, _NB):
        out_t = _concats[b](out_t, prev_t, gbs[b])
    return out_t.T


# speedup vs baseline: 1.0849x; 1.0178x over previous
import functools

import jax
import jax.numpy as jnp
import numpy as np
from jax import lax
from jax.experimental import pallas as pl
from jax.experimental.pallas import tpu as pltpu
from jax.experimental.pallas import tpu_sc as plsc

_N_NODES = 10000
_N_EDGES = 320000
_D_FEAT = 128
_SCALAR_DIM = 64
_R_PREV = 16
_OUT_DIM = _R_PREV + 2 * _SCALAR_DIM

_CHUNK = 128
_KSUB = 4
_SUPER = _KSUB * _CHUNK
_NC = 2
_NS = 16
_NW = _NC * _NS

_UNIT = 12800
_BLOCK_UNITS = [5, 5, 5, 5, 5]
_BLOCK_EDGES = [u * _UNIT for u in _BLOCK_UNITS]
_BLOCK_BASE = [sum(_BLOCK_EDGES[:b]) for b in range(len(_BLOCK_EDGES))]
_NB = len(_BLOCK_EDGES)

_INV_SQRT_FAN_IN = np.float32(1.0 / np.sqrt(np.float32(_D_FEAT)))


def _matmul_body(x_ref, w_ref, o_ref):
    o_ref[...] = jax.lax.dot_general(
        x_ref[...], w_ref[...],
        dimension_numbers=(((1,), (0,)), ((), ())),
        preferred_element_type=jnp.float32,
    ) * _INV_SQRT_FAN_IN


_node_linear = pl.pallas_call(
    _matmul_body,
    out_shape=jax.ShapeDtypeStruct((_N_NODES, _SCALAR_DIM), jnp.float32),
)


def _gather_body(base, n_supers, feat, esrc, edst, gb, isrc_v, idst_v, rsrc_v,
                 rdst_v, sem_idx, sem_g, sem_wr):
    wid = lax.axis_index("s") * _NC + lax.axis_index("c")

    def body(s, carry):
        sid = s * _NW + wid

        @pl.when(sid < n_supers)
        def _():
            r0 = sid * _SUPER
            e0 = base + r0
            cin = []
            for j in range(_KSUB):
                cin.append(pltpu.make_async_copy(
                    esrc.at[pl.ds(e0 + j * _CHUNK, _CHUNK)], isrc_v.at[j], sem_idx))
                cin.append(pltpu.make_async_copy(
                    edst.at[pl.ds(e0 + j * _CHUNK, _CHUNK)], idst_v.at[j], sem_idx))
            for c in cin:
                c.start()
            for c in cin:
                c.wait()
            cg = []
            for j in range(_KSUB):
                rows = pl.ds(j * _CHUNK, _CHUNK)
                cg.append(pltpu.make_async_copy(
                    feat.at[isrc_v.at[j]], rsrc_v.at[rows, :], sem_g))
                cg.append(pltpu.make_async_copy(
                    feat.at[idst_v.at[j]], rdst_v.at[rows, :], sem_g))
            for c in cg:
                c.start()
            for c in cg:
                c.wait()
            cw = [
                pltpu.make_async_copy(
                    rsrc_v, gb.at[pl.ds(r0, _SUPER), pl.ds(0, _SCALAR_DIM)], sem_wr),
                pltpu.make_async_copy(
                    rdst_v, gb.at[pl.ds(r0, _SUPER), pl.ds(_SCALAR_DIM, _SCALAR_DIM)], sem_wr),
            ]
            for c in cw:
                c.start()
            for c in cw:
                c.wait()

        return carry

    iters = (n_supers + _NW - 1) // _NW
    lax.fori_loop(0, iters, body, 0)


def _make_gather(b):
    n_supers = _BLOCK_EDGES[b] // _SUPER
    return functools.partial(
        pl.kernel,
        out_type=jax.ShapeDtypeStruct((_BLOCK_EDGES[b], 2 * _SCALAR_DIM), jnp.float32),
        mesh=plsc.VectorSubcoreMesh(
            core_axis_name="c", subcore_axis_name="s", num_cores=_NC, num_subcores=_NS
        ),
        scratch_types=[
            pltpu.VMEM((_KSUB, _CHUNK), jnp.int32),
            pltpu.VMEM((_KSUB, _CHUNK), jnp.int32),
            pltpu.VMEM((_SUPER, _SCALAR_DIM), jnp.float32),
            pltpu.VMEM((_SUPER, _SCALAR_DIM), jnp.float32),
            pltpu.SemaphoreType.DMA,
            pltpu.SemaphoreType.DMA,
            pltpu.SemaphoreType.DMA,
        ],
        compiler_params=pltpu.CompilerParams(use_tc_tiling_on_sc=False),
    )(functools.partial(_gather_body, _BLOCK_BASE[b], n_supers))


_gathers = [_make_gather(b) for b in range(_NB)]


_CB = 6400


def _concat_body(p_ref, b_ref, o_ref):
    o_ref[0:_R_PREV, :] = p_ref[...]
    o_ref[_R_PREV:_OUT_DIM, :] = b_ref[...].T


def _concat_body_alias(a_ref, p_ref, b_ref, o_ref):
    del a_ref
    o_ref[0:_R_PREV, :] = p_ref[...]
    o_ref[_R_PREV:_OUT_DIM, :] = b_ref[...].T


def _make_concat(b):
    base = _BLOCK_BASE[b] // _CB
    gsteps = _BLOCK_EDGES[b] // _CB
    if b == 0:
        return pl.pallas_call(
            _concat_body,
            grid=(gsteps,),
            in_specs=[
                pl.BlockSpec((_R_PREV, _CB), lambda i: (0, base + i)),
                pl.BlockSpec((_CB, 2 * _SCALAR_DIM), lambda i: (i, 0)),
            ],
            out_specs=pl.BlockSpec((_OUT_DIM, _CB), lambda i: (0, base + i)),
            out_shape=jax.ShapeDtypeStruct((_OUT_DIM, _N_EDGES), jnp.float32),
        )
    return pl.pallas_call(
        _concat_body_alias,
        grid=(gsteps,),
        in_specs=[
            pl.BlockSpec(memory_space=pl.ANY),
            pl.BlockSpec((_R_PREV, _CB), lambda i: (0, base + i)),
            pl.BlockSpec((_CB, 2 * _SCALAR_DIM), lambda i: (i, 0)),
        ],
        out_specs=pl.BlockSpec((_OUT_DIM, _CB), lambda i: (0, base + i)),
        out_shape=jax.ShapeDtypeStruct((_OUT_DIM, _N_EDGES), jnp.float32),
        input_output_aliases={0: 0},
    )


_concats = [_make_concat(b) for b in range(_NB)]


@jax.jit
def kernel(x, radial_weights_prev, edge_index, W):
    feat = _node_linear(x, W)
    edge_src = edge_index[1]
    edge_dst = edge_index[0]
    prev_t = radial_weights_prev.T
    gbs = [_gathers[b](feat, edge_src, edge_dst) for b in range(_NB)]
    out_t = _concats[0](prev_t, gbs[0])
    for b in range(1, _NB):
        out_t = _concats[b](out_t, prev_t, gbs[b])
    return out_t.T
